# Initial kernel scaffold; baseline (speedup 1.0000x reference)
import jax
import jax.numpy as jnp
from jax.experimental import pallas as pl


def kernel(x, edge_index, edge_attr, batch, W1_init, W1_root, b1, W2_init, W2_root, b2, lin1_w, lin1_b, lin2_w, lin2_b):
    raise NotImplementedError("write your pallas kernel here")


# speedup vs baseline: 8.3041x; 8.3041x over previous
import functools

import jax
import jax.numpy as jnp
from jax import lax
from jax.experimental import pallas as pl
from jax.experimental.pallas import tpu as pltpu
from jax.experimental.pallas import tpu_sc as plsc

N_ = 10000
E_ = 320000
G_ = 64
D_IN_ = 768
D_HID_ = 128

NC_ = 2
NS_ = 16
NW_ = NC_ * NS_
EPW_ = E_ // NW_
CH_ = 80
NCH_ = EPW_ // CH_
NP_ = 10240
RPT_ = NP_ // NS_

_MESH = dict(core_axis_name="c", subcore_axis_name="s")


def _sc_degree(col, ew, z1):
    mesh = plsc.VectorSubcoreMesh(**_MESH)

    @functools.partial(
        pl.kernel,
        out_type=jax.ShapeDtypeStruct((NC_, NP_), jnp.float32),
        mesh=mesh,
        scratch_types=[
            pltpu.VMEM((EPW_,), jnp.int32),
            pltpu.VMEM((EPW_,), jnp.float32),
            pltpu.VMEM((NP_,), jnp.float32),
            pltpu.VMEM((RPT_,), jnp.float32),
            pltpu.VMEM((RPT_,), jnp.float32),
            pltpu.VMEM_SHARED((NS_, NP_), jnp.float32),
        ],
        compiler_params=pltpu.CompilerParams(needs_layout_passes=False),
    )
    def k(col_hbm, ew_hbm, z_hbm, out_hbm, col_v, ew_v, acc_v, tmp_v, red_v,
          sh):
        c = lax.axis_index("c")
        s = lax.axis_index("s")
        wid = s * NC_ + c
        base = wid * EPW_
        pltpu.sync_copy(z_hbm, acc_v)
        pltpu.sync_copy(col_hbm.at[pl.ds(base, EPW_)], col_v)
        pltpu.sync_copy(ew_hbm.at[pl.ds(base, EPW_)], ew_v)

        def step(g, carry):
            i16 = col_v[pl.ds(g * 16, 16)]
            v16 = ew_v[pl.ds(g * 16, 16)]
            plsc.addupdate_scatter(acc_v, [i16], v16)
            return carry

        lax.fori_loop(0, EPW_ // 16, step, 0)
        pltpu.sync_copy(acc_v, sh.at[s])
        plsc.subcore_barrier()

        off = s * RPT_
        pltpu.sync_copy(sh.at[0, pl.ds(off, RPT_)], red_v)

        def red(t, carry):
            pltpu.sync_copy(sh.at[t, pl.ds(off, RPT_)], tmp_v)

            def addk(g, cc):
                sl = pl.ds(g * 16, 16)
                red_v[sl] = red_v[sl] + tmp_v[sl]
                return cc

            lax.fori_loop(0, RPT_ // 16, addk, 0)
            return carry

        lax.fori_loop(1, NS_, red, 0)
        pltpu.sync_copy(red_v, out_hbm.at[c, pl.ds(off, RPT_)])

    return k(col, ew, z1)


def _sc_gather_scatter(hs, row, col, ew, z128):
    mesh = plsc.VectorSubcoreMesh(**_MESH)

    @functools.partial(
        pl.kernel,
        out_type=jax.ShapeDtypeStruct((NC_ * NP_, D_HID_), jnp.float32),
        mesh=mesh,
        scratch_types=[
            pltpu.VMEM((CH_,), jnp.int32),
            pltpu.VMEM((CH_,), jnp.int32),
            pltpu.VMEM((CH_,), jnp.float32),
            pltpu.VMEM((CH_, D_HID_), jnp.float32),
            pltpu.VMEM_SHARED((NP_, D_HID_), jnp.float32),
            pltpu.SemaphoreType.DMA,
        ],
        compiler_params=pltpu.CompilerParams(needs_layout_passes=False),
    )
    def k(hs_hbm, row_hbm, col_hbm, ew_hbm, z_hbm, out_hbm,
          row_v, col_v, ew_v, rows_v, acc_sh, sem):
        c = lax.axis_index("c")
        s = lax.axis_index("s")
        wid = s * NC_ + c
        pltpu.sync_copy(z_hbm, acc_sh.at[pl.ds(s * RPT_, RPT_)])
        plsc.subcore_barrier()

        base = wid * EPW_

        def chunk(j, carry):
            off = base + j * CH_
            pltpu.sync_copy(row_hbm.at[pl.ds(off, CH_)], row_v)
            pltpu.sync_copy(col_hbm.at[pl.ds(off, CH_)], col_v)
            pltpu.sync_copy(ew_hbm.at[pl.ds(off, CH_)], ew_v)
            pltpu.async_copy(hs_hbm.at[row_v], rows_v, sem).wait()

            def scale(g, cc):
                ew16 = ew_v[pl.ds(g * 16, 16)]
                for i in range(16):
                    w = ew16[i]
                    e = g * 16 + i
                    for fg in range(D_HID_ // 16):
                        sl = pl.ds(fg * 16, 16)
                        rows_v[e, sl] = rows_v[e, sl] * w
                return cc

            lax.fori_loop(0, CH_ // 16, scale, 0)
            pltpu.sync_copy(rows_v, acc_sh.at[col_v], add=True)
            return carry

        lax.fori_loop(0, NCH_, chunk, 0)
        plsc.subcore_barrier()
        pltpu.sync_copy(acc_sh.at[pl.ds(s * RPT_, RPT_)],
                        out_hbm.at[pl.ds(c * NP_ + s * RPT_, RPT_)])

    return k(hs, row, col, ew, z128)


_BN = 1000


def _tc1_body(x_ref, wi_ref, wr_ref, degp_ref, hs_ref, r_ref, dinv_ref):
    xb = x_ref[...]
    h = jnp.dot(xb, wi_ref[...], preferred_element_type=jnp.float32)
    r = jnp.dot(xb, wr_ref[...], preferred_element_type=jnp.float32)
    d = degp_ref[0] + degp_ref[1]
    safe = jnp.where(d > 0, d, 1.0)
    dinv = jnp.where(d > 0, lax.rsqrt(safe), 0.0)
    hs_ref[...] = h * dinv
    r_ref[...] = r
    dinv_ref[...] = jnp.broadcast_to(dinv, (_BN, 8))


def _tc_layer1(x, w1i, w1r, degp3):
    grid = (N_ // _BN,)
    return pl.pallas_call(
        _tc1_body,
        grid=grid,
        in_specs=[
            pl.BlockSpec((_BN, D_IN_), lambda i: (i, 0)),
            pl.BlockSpec((D_IN_, D_HID_), lambda i: (0, 0)),
            pl.BlockSpec((D_IN_, D_HID_), lambda i: (0, 0)),
            pl.BlockSpec((NC_, _BN, 1), lambda i: (0, i, 0)),
        ],
        out_specs=[
            pl.BlockSpec((_BN, D_HID_), lambda i: (i, 0)),
            pl.BlockSpec((_BN, D_HID_), lambda i: (i, 0)),
            pl.BlockSpec((_BN, 8), lambda i: (i, 0)),
        ],
        out_shape=[
            jax.ShapeDtypeStruct((N_, D_HID_), jnp.float32),
            jax.ShapeDtypeStruct((N_, D_HID_), jnp.float32),
            jax.ShapeDtypeStruct((N_, 8), jnp.float32),
        ],
    )(x, w1i, w1r, degp3)


def _tc2_body(acc_ref, r1_ref, dinv_ref, b1_ref, wi_ref, wr_ref,
              hs2_ref, r2_ref):
    a = acc_ref[0] + acc_ref[1]
    dinv = dinv_ref[...][:, 0:1]
    h1 = jnp.maximum(a * dinv + r1_ref[...] + b1_ref[...], 0.0)
    hs2_ref[...] = jnp.dot(h1, wi_ref[...],
                           preferred_element_type=jnp.float32) * dinv
    r2_ref[...] = jnp.dot(h1, wr_ref[...], preferred_element_type=jnp.float32)


def _tc_layer2(acc1, r1, dinv8, b1, w2i, w2r):
    grid = (N_ // _BN,)
    return pl.pallas_call(
        _tc2_body,
        grid=grid,
        in_specs=[
            pl.BlockSpec((NC_, _BN, D_HID_), lambda i: (0, i, 0)),
            pl.BlockSpec((_BN, D_HID_), lambda i: (i, 0)),
            pl.BlockSpec((_BN, 8), lambda i: (i, 0)),
            pl.BlockSpec((1, D_HID_), lambda i: (0, 0)),
            pl.BlockSpec((D_HID_, D_HID_), lambda i: (0, 0)),
            pl.BlockSpec((D_HID_, D_HID_), lambda i: (0, 0)),
        ],
        out_specs=[
            pl.BlockSpec((_BN, D_HID_), lambda i: (i, 0)),
            pl.BlockSpec((_BN, D_HID_), lambda i: (i, 0)),
        ],
        out_shape=[
            jax.ShapeDtypeStruct((N_, D_HID_), jnp.float32),
            jax.ShapeDtypeStruct((N_, D_HID_), jnp.float32),
        ],
    )(acc1, r1, dinv8, b1, w2i, w2r)


def _tc3_body(acc_ref, r2_ref, dinv_ref, b2_ref, batch_ref,
              l1w_ref, l1b_ref, l2w_ref, l2b_ref, out_ref):
    a = acc_ref[0] + acc_ref[1]
    dinv = dinv_ref[...][:, 0:1]
    node = jnp.maximum(a * dinv + r2_ref[...] + b2_ref[...], 0.0)
    gids = lax.broadcasted_iota(jnp.int32, (G_, N_), 0)
    oh = (gids == batch_ref[...]).astype(jnp.float32)
    sums = jnp.dot(oh, node, preferred_element_type=jnp.float32)
    cnt = jnp.sum(oh, axis=1, keepdims=True)
    graph = sums / jnp.maximum(cnt, 1.0)
    t = jnp.dot(graph, l1w_ref[...],
                preferred_element_type=jnp.float32) + l1b_ref[...]
    hh = jnp.where(t > 0, t, jnp.exp(t) - 1.0)
    out_ref[...] = jnp.dot(hh, l2w_ref[...],
                           preferred_element_type=jnp.float32) + l2b_ref[...]


def _tc_head(acc2, r2, dinv8, b2, batch2d, l1w, l1b, l2w, l2b):
    return pl.pallas_call(
        _tc3_body,
        grid=(1,),
        in_specs=[
            pl.BlockSpec((NC_, N_, D_HID_), lambda i: (0, 0, 0)),
            pl.BlockSpec((N_, D_HID_), lambda i: (0, 0)),
            pl.BlockSpec((N_, 8), lambda i: (0, 0)),
            pl.BlockSpec((1, D_HID_), lambda i: (0, 0)),
            pl.BlockSpec((1, N_), lambda i: (0, 0)),
            pl.BlockSpec((D_HID_, 32), lambda i: (0, 0)),
            pl.BlockSpec((1, 32), lambda i: (0, 0)),
            pl.BlockSpec((32, 2), lambda i: (0, 0)),
            pl.BlockSpec((1, 2), lambda i: (0, 0)),
        ],
        out_specs=pl.BlockSpec((G_, 2), lambda i: (0, 0)),
        out_shape=jax.ShapeDtypeStruct((G_, 2), jnp.float32),
    )(acc2, r2, dinv8, b2, batch2d, l1w, l1b, l2w, l2b)


def kernel(x, edge_index, edge_attr, batch, W1_init, W1_root, b1,
           W2_init, W2_root, b2, lin1_w, lin1_b, lin2_w, lin2_b):
    row = edge_index[0]
    col = edge_index[1]
    ew = edge_attr.reshape(E_)
    z1 = jnp.zeros((NP_,), jnp.float32)
    z128 = jnp.zeros((RPT_, D_HID_), jnp.float32)

    degp = _sc_degree(col, ew, z1).reshape(NC_, NP_, 1)
    hs1, r1, dinv8 = _tc_layer1(x, W1_init, W1_root, degp)
    acc1 = _sc_gather_scatter(hs1, row, col, ew, z128).reshape(NC_, NP_, D_HID_)
    hs2, r2 = _tc_layer2(acc1, r1, dinv8, b1.reshape(1, D_HID_),
                         W2_init, W2_root)
    acc2 = _sc_gather_scatter(hs2, row, col, ew, z128).reshape(NC_, NP_, D_HID_)
    pred = _tc_head(acc2, r2, dinv8, b2.reshape(1, D_HID_),
                    batch.reshape(1, N_), lin1_w, lin1_b.reshape(1, 32),
                    lin2_w, lin2_b.reshape(1, 2))
    return pred


# speedup vs baseline: 15.8223x; 1.9054x over previous
import functools

import jax
import jax.numpy as jnp
from jax import lax
from jax.experimental import pallas as pl
from jax.experimental.pallas import tpu as pltpu
from jax.experimental.pallas import tpu_sc as plsc

N_ = 10000
E_ = 320000
G_ = 64
D_IN_ = 768
D_HID_ = 128

NC_ = 2
NS_ = 16
NW_ = NC_ * NS_
EPW_ = E_ // NW_
CH_ = 80
NCH_ = EPW_ // CH_
NP_ = 10240
RPT_ = NP_ // NS_

_MESH = dict(core_axis_name="c", subcore_axis_name="s")


def _sc_degree(col, ew, z1):
    mesh = plsc.VectorSubcoreMesh(**_MESH)

    @functools.partial(
        pl.kernel,
        out_type=jax.ShapeDtypeStruct((NC_, NP_), jnp.float32),
        mesh=mesh,
        scratch_types=[
            pltpu.VMEM((EPW_,), jnp.int32),
            pltpu.VMEM((EPW_,), jnp.float32),
            pltpu.VMEM((NP_,), jnp.float32),
            pltpu.VMEM((RPT_,), jnp.float32),
            pltpu.VMEM((RPT_,), jnp.float32),
            pltpu.VMEM_SHARED((NS_, NP_), jnp.float32),
        ],
        compiler_params=pltpu.CompilerParams(needs_layout_passes=False),
    )
    def k(col_hbm, ew_hbm, z_hbm, out_hbm, col_v, ew_v, acc_v, tmp_v, red_v,
          sh):
        c = lax.axis_index("c")
        s = lax.axis_index("s")
        wid = s * NC_ + c
        base = wid * EPW_
        pltpu.sync_copy(z_hbm, acc_v)
        pltpu.sync_copy(col_hbm.at[pl.ds(base, EPW_)], col_v)
        pltpu.sync_copy(ew_hbm.at[pl.ds(base, EPW_)], ew_v)

        def step(g, carry):
            i16 = col_v[pl.ds(g * 16, 16)]
            v16 = ew_v[pl.ds(g * 16, 16)]
            plsc.addupdate_scatter(acc_v, [i16], v16)
            return carry

        lax.fori_loop(0, EPW_ // 16, step, 0)
        pltpu.sync_copy(acc_v, sh.at[s])
        plsc.subcore_barrier()

        off = s * RPT_
        pltpu.sync_copy(sh.at[0, pl.ds(off, RPT_)], red_v)

        def red(t, carry):
            pltpu.sync_copy(sh.at[t, pl.ds(off, RPT_)], tmp_v)

            def addk(g, cc):
                sl = pl.ds(g * 16, 16)
                red_v[sl] = red_v[sl] + tmp_v[sl]
                return cc

            lax.fori_loop(0, RPT_ // 16, addk, 0)
            return carry

        lax.fori_loop(1, NS_, red, 0)
        pltpu.sync_copy(red_v, out_hbm.at[c, pl.ds(off, RPT_)])

    return k(col, ew, z1)


NBUF_ = 2


def _sc_gather_scatter(hs, rc4, ew3, z128):
    mesh = plsc.VectorSubcoreMesh(**_MESH)

    @functools.partial(
        pl.kernel,
        out_type=jax.ShapeDtypeStruct((NC_ * NP_, D_HID_), jnp.float32),
        mesh=mesh,
        scratch_types=(
            [
                pltpu.VMEM((NBUF_, 2, CH_), jnp.int32),
                pltpu.VMEM((NCH_, CH_), jnp.float32),
            ]
            + [pltpu.VMEM((CH_, D_HID_), jnp.float32) for _ in range(NBUF_)]
            + [pltpu.VMEM_SHARED((NP_, D_HID_), jnp.float32)]
            + [pltpu.SemaphoreType.DMA for _ in range(NBUF_)]
        ),
        compiler_params=pltpu.CompilerParams(needs_layout_passes=False),
    )
    def k(hs_hbm, rc_hbm, ew_hbm, z_hbm, out_hbm, rc_v, ew_v, *rest):
        bufs = rest[:NBUF_]
        acc_sh = rest[NBUF_]
        gsem = rest[NBUF_ + 1:]
        c = lax.axis_index("c")
        s = lax.axis_index("s")
        wid = s * NC_ + c
        pltpu.sync_copy(z_hbm, acc_sh.at[pl.ds(s * RPT_, RPT_)])
        pltpu.sync_copy(ew_hbm.at[wid], ew_v)
        plsc.subcore_barrier()

        def gat(j, b):
            pltpu.sync_copy(rc_hbm.at[wid, j], rc_v.at[b])
            pltpu.async_copy(hs_hbm.at[rc_v.at[b, 0]], bufs[b], gsem[b])

        def gwait(j, b):
            pltpu.make_async_copy(hs_hbm.at[rc_v.at[b, 0]], bufs[b],
                                  gsem[b]).wait()

        def consume(jj, b):
            def scale(g, cc):
                ew16 = ew_v[jj, pl.ds(g * 16, 16)]
                for i in range(16):
                    w = ew16[i]
                    e = g * 16 + i
                    for fg in range(D_HID_ // 16):
                        sl = pl.ds(fg * 16, 16)
                        bufs[b][e, sl] = bufs[b][e, sl] * w
                return cc

            lax.fori_loop(0, CH_ // 16, scale, 0)
            pltpu.sync_copy(bufs[b], acc_sh.at[rc_v.at[b, 1]], add=True)

        gat(0, 0)

        def rounds(t, carry):
            j0 = t * NBUF_
            for b in range(NBUF_):
                jj = j0 + b
                gat(jj + 1, (b + 1) % NBUF_)
                gwait(jj, b)
                consume(jj, b)
            return carry

        lax.fori_loop(0, (NCH_ - 1) // NBUF_, rounds, 0)
        gwait(NCH_ - 1, (NCH_ - 1) % NBUF_)
        consume(NCH_ - 1, (NCH_ - 1) % NBUF_)
        plsc.subcore_barrier()
        pltpu.sync_copy(acc_sh.at[pl.ds(s * RPT_, RPT_)],
                        out_hbm.at[pl.ds(c * NP_ + s * RPT_, RPT_)])

    return k(hs, rc4, ew3, z128)


_BN = 1000


def _tc1_body(x_ref, wi_ref, wr_ref, degp_ref, hs_ref, r_ref, dinv_ref):
    xb = x_ref[...]
    h = jnp.dot(xb, wi_ref[...], preferred_element_type=jnp.float32)
    r = jnp.dot(xb, wr_ref[...], preferred_element_type=jnp.float32)
    d = degp_ref[0] + degp_ref[1]
    safe = jnp.where(d > 0, d, 1.0)
    dinv = jnp.where(d > 0, lax.rsqrt(safe), 0.0)
    hs_ref[...] = h * dinv
    r_ref[...] = r
    dinv_ref[...] = jnp.broadcast_to(dinv, (_BN, 8))


def _tc_layer1(x, w1i, w1r, degp3):
    grid = (N_ // _BN,)
    return pl.pallas_call(
        _tc1_body,
        grid=grid,
        in_specs=[
            pl.BlockSpec((_BN, D_IN_), lambda i: (i, 0)),
            pl.BlockSpec((D_IN_, D_HID_), lambda i: (0, 0)),
            pl.BlockSpec((D_IN_, D_HID_), lambda i: (0, 0)),
            pl.BlockSpec((NC_, _BN, 1), lambda i: (0, i, 0)),
        ],
        out_specs=[
            pl.BlockSpec((_BN, D_HID_), lambda i: (i, 0)),
            pl.BlockSpec((_BN, D_HID_), lambda i: (i, 0)),
            pl.BlockSpec((_BN, 8), lambda i: (i, 0)),
        ],
        out_shape=[
            jax.ShapeDtypeStruct((N_, D_HID_), jnp.float32),
            jax.ShapeDtypeStruct((N_, D_HID_), jnp.float32),
            jax.ShapeDtypeStruct((N_, 8), jnp.float32),
        ],
    )(x, w1i, w1r, degp3)


def _tc2_body(acc_ref, r1_ref, dinv_ref, b1_ref, wi_ref, wr_ref,
              hs2_ref, r2_ref):
    a = acc_ref[0] + acc_ref[1]
    dinv = dinv_ref[...][:, 0:1]
    h1 = jnp.maximum(a * dinv + r1_ref[...] + b1_ref[...], 0.0)
    hs2_ref[...] = jnp.dot(h1, wi_ref[...],
                           preferred_element_type=jnp.float32) * dinv
    r2_ref[...] = jnp.dot(h1, wr_ref[...], preferred_element_type=jnp.float32)


def _tc_layer2(acc1, r1, dinv8, b1, w2i, w2r):
    grid = (N_ // _BN,)
    return pl.pallas_call(
        _tc2_body,
        grid=grid,
        in_specs=[
            pl.BlockSpec((NC_, _BN, D_HID_), lambda i: (0, i, 0)),
            pl.BlockSpec((_BN, D_HID_), lambda i: (i, 0)),
            pl.BlockSpec((_BN, 8), lambda i: (i, 0)),
            pl.BlockSpec((1, D_HID_), lambda i: (0, 0)),
            pl.BlockSpec((D_HID_, D_HID_), lambda i: (0, 0)),
            pl.BlockSpec((D_HID_, D_HID_), lambda i: (0, 0)),
        ],
        out_specs=[
            pl.BlockSpec((_BN, D_HID_), lambda i: (i, 0)),
            pl.BlockSpec((_BN, D_HID_), lambda i: (i, 0)),
        ],
        out_shape=[
            jax.ShapeDtypeStruct((N_, D_HID_), jnp.float32),
            jax.ShapeDtypeStruct((N_, D_HID_), jnp.float32),
        ],
    )(acc1, r1, dinv8, b1, w2i, w2r)


def _tc3_body(acc_ref, r2_ref, dinv_ref, b2_ref, batch_ref,
              l1w_ref, l1b_ref, l2w_ref, l2b_ref, out_ref):
    a = acc_ref[0] + acc_ref[1]
    dinv = dinv_ref[...][:, 0:1]
    node = jnp.maximum(a * dinv + r2_ref[...] + b2_ref[...], 0.0)
    gids = lax.broadcasted_iota(jnp.int32, (G_, N_), 0)
    oh = (gids == batch_ref[...]).astype(jnp.float32)
    sums = jnp.dot(oh, node, preferred_element_type=jnp.float32)
    cnt = jnp.sum(oh, axis=1, keepdims=True)
    graph = sums / jnp.maximum(cnt, 1.0)
    t = jnp.dot(graph, l1w_ref[...],
                preferred_element_type=jnp.float32) + l1b_ref[...]
    hh = jnp.where(t > 0, t, jnp.exp(t) - 1.0)
    out_ref[...] = jnp.dot(hh, l2w_ref[...],
                           preferred_element_type=jnp.float32) + l2b_ref[...]


def _tc_head(acc2, r2, dinv8, b2, batch2d, l1w, l1b, l2w, l2b):
    return pl.pallas_call(
        _tc3_body,
        grid=(1,),
        in_specs=[
            pl.BlockSpec((NC_, N_, D_HID_), lambda i: (0, 0, 0)),
            pl.BlockSpec((N_, D_HID_), lambda i: (0, 0)),
            pl.BlockSpec((N_, 8), lambda i: (0, 0)),
            pl.BlockSpec((1, D_HID_), lambda i: (0, 0)),
            pl.BlockSpec((1, N_), lambda i: (0, 0)),
            pl.BlockSpec((D_HID_, 32), lambda i: (0, 0)),
            pl.BlockSpec((1, 32), lambda i: (0, 0)),
            pl.BlockSpec((32, 2), lambda i: (0, 0)),
            pl.BlockSpec((1, 2), lambda i: (0, 0)),
        ],
        out_specs=pl.BlockSpec((G_, 2), lambda i: (0, 0)),
        out_shape=jax.ShapeDtypeStruct((G_, 2), jnp.float32),
    )(acc2, r2, dinv8, b2, batch2d, l1w, l1b, l2w, l2b)


def kernel(x, edge_index, edge_attr, batch, W1_init, W1_root, b1,
           W2_init, W2_root, b2, lin1_w, lin1_b, lin2_w, lin2_b):
    row = edge_index[0]
    col = edge_index[1]
    ew = edge_attr.reshape(E_)
    rc4 = jnp.stack([row.reshape(NW_, NCH_, CH_),
                     col.reshape(NW_, NCH_, CH_)], axis=2)
    ew3 = ew.reshape(NW_, NCH_, CH_)
    z1 = jnp.zeros((NP_,), jnp.float32)
    z128 = jnp.zeros((RPT_, D_HID_), jnp.float32)

    degp = _sc_degree(col, ew, z1).reshape(NC_, NP_, 1)
    hs1, r1, dinv8 = _tc_layer1(x, W1_init, W1_root, degp)
    acc1 = _sc_gather_scatter(hs1, rc4, ew3,
                              z128).reshape(NC_, NP_, D_HID_)
    hs2, r2 = _tc_layer2(acc1, r1, dinv8, b1.reshape(1, D_HID_),
                         W2_init, W2_root)
    acc2 = _sc_gather_scatter(hs2, rc4, ew3,
                              z128).reshape(NC_, NP_, D_HID_)
    pred = _tc_head(acc2, r2, dinv8, b2.reshape(1, D_HID_),
                    batch.reshape(1, N_), lin1_w, lin1_b.reshape(1, 32),
                    lin2_w, lin2_b.reshape(1, 2))
    return pred


# speedup vs baseline: 15.8356x; 1.0008x over previous
import functools

import jax
import jax.numpy as jnp
from jax import lax
from jax.experimental import pallas as pl
from jax.experimental.pallas import tpu as pltpu
from jax.experimental.pallas import tpu_sc as plsc

N_ = 10000
E_ = 320000
G_ = 64
D_IN_ = 768
D_HID_ = 128

NC_ = 2
NS_ = 16
NW_ = NC_ * NS_
EPW_ = E_ // NW_
CH_ = 80
NCH_ = EPW_ // CH_
NP_ = 10240
RPT_ = NP_ // NS_

_MESH = dict(core_axis_name="c", subcore_axis_name="s")


def _sc_degree(col, ew, z1):
    mesh = plsc.VectorSubcoreMesh(**_MESH)

    @functools.partial(
        pl.kernel,
        out_type=jax.ShapeDtypeStruct((NC_, NP_), jnp.float32),
        mesh=mesh,
        scratch_types=[
            pltpu.VMEM((EPW_,), jnp.int32),
            pltpu.VMEM((EPW_,), jnp.float32),
            pltpu.VMEM((NP_,), jnp.float32),
            pltpu.VMEM((RPT_,), jnp.float32),
            pltpu.VMEM((RPT_,), jnp.float32),
            pltpu.VMEM_SHARED((NS_, NP_), jnp.float32),
        ],
        compiler_params=pltpu.CompilerParams(needs_layout_passes=False),
    )
    def k(col_hbm, ew_hbm, z_hbm, out_hbm, col_v, ew_v, acc_v, tmp_v, red_v,
          sh):
        c = lax.axis_index("c")
        s = lax.axis_index("s")
        wid = s * NC_ + c
        base = wid * EPW_
        pltpu.sync_copy(z_hbm, acc_v)
        pltpu.sync_copy(col_hbm.at[pl.ds(base, EPW_)], col_v)
        pltpu.sync_copy(ew_hbm.at[pl.ds(base, EPW_)], ew_v)

        def step(g, carry):
            i16 = col_v[pl.ds(g * 16, 16)]
            v16 = ew_v[pl.ds(g * 16, 16)]
            plsc.addupdate_scatter(acc_v, [i16], v16)
            return carry

        lax.fori_loop(0, EPW_ // 16, step, 0)
        pltpu.sync_copy(acc_v, sh.at[s])
        plsc.subcore_barrier()

        off = s * RPT_
        pltpu.sync_copy(sh.at[0, pl.ds(off, RPT_)], red_v)

        def red(t, carry):
            pltpu.sync_copy(sh.at[t, pl.ds(off, RPT_)], tmp_v)

            def addk(g, cc):
                sl = pl.ds(g * 16, 16)
                red_v[sl] = red_v[sl] + tmp_v[sl]
                return cc

            lax.fori_loop(0, RPT_ // 16, addk, 0)
            return carry

        lax.fori_loop(1, NS_, red, 0)
        pltpu.sync_copy(red_v, out_hbm.at[c, pl.ds(off, RPT_)])

    return k(col, ew, z1)


NBUF_ = 3


def _sc_gather_scatter(hs, rc4, ew3, z128):
    mesh = plsc.VectorSubcoreMesh(**_MESH)

    @functools.partial(
        pl.kernel,
        out_type=jax.ShapeDtypeStruct((NC_ * NP_, D_HID_), jnp.float32),
        mesh=mesh,
        scratch_types=(
            [
                pltpu.VMEM((NBUF_, 2, CH_), jnp.int32),
                pltpu.VMEM((NCH_, CH_), jnp.float32),
            ]
            + [pltpu.VMEM((CH_, D_HID_), jnp.float32) for _ in range(NBUF_)]
            + [pltpu.VMEM_SHARED((NP_, D_HID_), jnp.float32)]
            + [pltpu.SemaphoreType.DMA for _ in range(NBUF_)]
        ),
        compiler_params=pltpu.CompilerParams(needs_layout_passes=False),
    )
    def k(hs_hbm, rc_hbm, ew_hbm, z_hbm, out_hbm, rc_v, ew_v, *rest):
        bufs = rest[:NBUF_]
        acc_sh = rest[NBUF_]
        gsem = rest[NBUF_ + 1:]
        c = lax.axis_index("c")
        s = lax.axis_index("s")
        wid = s * NC_ + c
        pltpu.sync_copy(z_hbm, acc_sh.at[pl.ds(s * RPT_, RPT_)])
        pltpu.sync_copy(ew_hbm.at[wid], ew_v)
        plsc.subcore_barrier()

        def gat(j, b):
            pltpu.sync_copy(rc_hbm.at[wid, j], rc_v.at[b])
            pltpu.async_copy(hs_hbm.at[rc_v.at[b, 0]], bufs[b], gsem[b])

        def gwait(j, b):
            pltpu.make_async_copy(hs_hbm.at[rc_v.at[b, 0]], bufs[b],
                                  gsem[b]).wait()

        def consume(jj, b):
            def scale(g, cc):
                ew16 = ew_v[jj, pl.ds(g * 16, 16)]
                for i in range(16):
                    w = ew16[i]
                    e = g * 16 + i
                    for fg in range(D_HID_ // 16):
                        sl = pl.ds(fg * 16, 16)
                        bufs[b][e, sl] = bufs[b][e, sl] * w
                return cc

            lax.fori_loop(0, CH_ // 16, scale, 0)
            pltpu.sync_copy(bufs[b], acc_sh.at[rc_v.at[b, 1]], add=True)

        gat(0, 0)
        gat(1, 1)

        def rounds(t, carry):
            j0 = t * NBUF_
            for b in range(NBUF_):
                jj = j0 + b
                gat(jj + 2, (b + 2) % NBUF_)
                gwait(jj, b)
                consume(jj, b)
            return carry

        lax.fori_loop(0, (NCH_ - 2) // NBUF_, rounds, 0)
        for jj in (NCH_ - 2, NCH_ - 1):
            gwait(jj, jj % NBUF_)
            consume(jj, jj % NBUF_)
        plsc.subcore_barrier()
        pltpu.sync_copy(acc_sh.at[pl.ds(s * RPT_, RPT_)],
                        out_hbm.at[pl.ds(c * NP_ + s * RPT_, RPT_)])

    return k(hs, rc4, ew3, z128)


_BN = 1000


def _tc1_body(x_ref, wi_ref, wr_ref, degp_ref, hs_ref, r_ref, dinv_ref):
    xb = x_ref[...]
    h = jnp.dot(xb, wi_ref[...], preferred_element_type=jnp.float32)
    r = jnp.dot(xb, wr_ref[...], preferred_element_type=jnp.float32)
    d = degp_ref[0] + degp_ref[1]
    safe = jnp.where(d > 0, d, 1.0)
    dinv = jnp.where(d > 0, 1.0 / jnp.sqrt(safe), 0.0)
    hs_ref[...] = h * dinv
    r_ref[...] = r
    dinv_ref[...] = jnp.broadcast_to(dinv, (_BN, 8))


def _tc_layer1(x, w1i, w1r, degp3):
    grid = (N_ // _BN,)
    return pl.pallas_call(
        _tc1_body,
        grid=grid,
        in_specs=[
            pl.BlockSpec((_BN, D_IN_), lambda i: (i, 0)),
            pl.BlockSpec((D_IN_, D_HID_), lambda i: (0, 0)),
            pl.BlockSpec((D_IN_, D_HID_), lambda i: (0, 0)),
            pl.BlockSpec((NC_, _BN, 1), lambda i: (0, i, 0)),
        ],
        out_specs=[
            pl.BlockSpec((_BN, D_HID_), lambda i: (i, 0)),
            pl.BlockSpec((_BN, D_HID_), lambda i: (i, 0)),
            pl.BlockSpec((_BN, 8), lambda i: (i, 0)),
        ],
        out_shape=[
            jax.ShapeDtypeStruct((N_, D_HID_), jnp.float32),
            jax.ShapeDtypeStruct((N_, D_HID_), jnp.float32),
            jax.ShapeDtypeStruct((N_, 8), jnp.float32),
        ],
    )(x, w1i, w1r, degp3)


def _tc2_body(acc_ref, r1_ref, dinv_ref, b1_ref, wi_ref, wr_ref,
              hs2_ref, r2_ref):
    a = acc_ref[0] + acc_ref[1]
    dinv = dinv_ref[...][:, 0:1]
    h1 = jnp.maximum(a * dinv + r1_ref[...] + b1_ref[...], 0.0)
    hs2_ref[...] = jnp.dot(h1, wi_ref[...],
                           preferred_element_type=jnp.float32) * dinv
    r2_ref[...] = jnp.dot(h1, wr_ref[...], preferred_element_type=jnp.float32)


def _tc_layer2(acc1, r1, dinv8, b1, w2i, w2r):
    grid = (N_ // _BN,)
    return pl.pallas_call(
        _tc2_body,
        grid=grid,
        in_specs=[
            pl.BlockSpec((NC_, _BN, D_HID_), lambda i: (0, i, 0)),
            pl.BlockSpec((_BN, D_HID_), lambda i: (i, 0)),
            pl.BlockSpec((_BN, 8), lambda i: (i, 0)),
            pl.BlockSpec((1, D_HID_), lambda i: (0, 0)),
            pl.BlockSpec((D_HID_, D_HID_), lambda i: (0, 0)),
            pl.BlockSpec((D_HID_, D_HID_), lambda i: (0, 0)),
        ],
        out_specs=[
            pl.BlockSpec((_BN, D_HID_), lambda i: (i, 0)),
            pl.BlockSpec((_BN, D_HID_), lambda i: (i, 0)),
        ],
        out_shape=[
            jax.ShapeDtypeStruct((N_, D_HID_), jnp.float32),
            jax.ShapeDtypeStruct((N_, D_HID_), jnp.float32),
        ],
    )(acc1, r1, dinv8, b1, w2i, w2r)


def _tc3_body(acc_ref, r2_ref, dinv_ref, b2_ref, batch_ref,
              l1w_ref, l1b_ref, l2w_ref, l2b_ref, out_ref):
    a = acc_ref[0] + acc_ref[1]
    dinv = dinv_ref[...][:, 0:1]
    node = jnp.maximum(a * dinv + r2_ref[...] + b2_ref[...], 0.0)
    gids = lax.broadcasted_iota(jnp.int32, (G_, N_), 0)
    oh = (gids == batch_ref[...]).astype(jnp.float32)
    sums = jnp.dot(oh, node, preferred_element_type=jnp.float32)
    cnt = jnp.sum(oh, axis=1, keepdims=True)
    graph = sums / jnp.maximum(cnt, 1.0)
    t = jnp.dot(graph, l1w_ref[...],
                preferred_element_type=jnp.float32) + l1b_ref[...]
    hh = jnp.where(t > 0, t, jnp.exp(t) - 1.0)
    out_ref[...] = jnp.dot(hh, l2w_ref[...],
                           preferred_element_type=jnp.float32) + l2b_ref[...]


def _tc_head(acc2, r2, dinv8, b2, batch2d, l1w, l1b, l2w, l2b):
    return pl.pallas_call(
        _tc3_body,
        grid=(1,),
        in_specs=[
            pl.BlockSpec((NC_, N_, D_HID_), lambda i: (0, 0, 0)),
            pl.BlockSpec((N_, D_HID_), lambda i: (0, 0)),
            pl.BlockSpec((N_, 8), lambda i: (0, 0)),
            pl.BlockSpec((1, D_HID_), lambda i: (0, 0)),
            pl.BlockSpec((1, N_), lambda i: (0, 0)),
            pl.BlockSpec((D_HID_, 32), lambda i: (0, 0)),
            pl.BlockSpec((1, 32), lambda i: (0, 0)),
            pl.BlockSpec((32, 2), lambda i: (0, 0)),
            pl.BlockSpec((1, 2), lambda i: (0, 0)),
        ],
        out_specs=pl.BlockSpec((G_, 2), lambda i: (0, 0)),
        out_shape=jax.ShapeDtypeStruct((G_, 2), jnp.float32),
    )(acc2, r2, dinv8, b2, batch2d, l1w, l1b, l2w, l2b)


def kernel(x, edge_index, edge_attr, batch, W1_init, W1_root, b1,
           W2_init, W2_root, b2, lin1_w, lin1_b, lin2_w, lin2_b):
    row = edge_index[0]
    col = edge_index[1]
    ew = edge_attr.reshape(E_)
    rc4 = jnp.stack([row.reshape(NW_, NCH_, CH_),
                     col.reshape(NW_, NCH_, CH_)], axis=2)
    ew3 = ew.reshape(NW_, NCH_, CH_)
    z1 = jnp.zeros((NP_,), jnp.float32)
    z128 = jnp.zeros((RPT_, D_HID_), jnp.float32)

    degp = _sc_degree(col, ew, z1).reshape(NC_, NP_, 1)
    hs1, r1, dinv8 = _tc_layer1(x, W1_init, W1_root, degp)
    acc1 = _sc_gather_scatter(hs1, rc4, ew3,
                              z128).reshape(NC_, NP_, D_HID_)
    hs2, r2 = _tc_layer2(acc1, r1, dinv8, b1.reshape(1, D_HID_),
                         W2_init, W2_root)
    acc2 = _sc_gather_scatter(hs2, rc4, ew3,
                              z128).reshape(NC_, NP_, D_HID_)
    pred = _tc_head(acc2, r2, dinv8, b2.reshape(1, D_HID_),
                    batch.reshape(1, N_), lin1_w, lin1_b.reshape(1, 32),
                    lin2_w, lin2_b.reshape(1, 2))
    return pred
